# baseline (speedup 1.0000x reference)
import jax
import jax.numpy as jnp
import numpy as np
from jax.experimental import pallas as pl
from jax.experimental.pallas import tpu as pltpu

jax.config.update("jax_enable_x64", True)

_I0 = np.int32(0)


def _xw_kernel(x_ref, w_ref, o_ref):
    o_ref[...] = jnp.dot(x_ref[...], w_ref[...],
                         preferred_element_type=jnp.float32)


def _spmm_relu_kernel(s_ref, xw_ref, o_ref, s_out_ref):
    s = s_ref[...]
    acc = jnp.dot(s, xw_ref[...], preferred_element_type=jnp.float32)
    o_ref[...] = jnp.maximum(acc, 0.0)
    s_out_ref[...] = s


def kernel(x, support, weight):
    n, d_in = x.shape
    d_out = weight.shape[1]

    xw = pl.pallas_call(
        _xw_kernel,
        out_shape=jax.ShapeDtypeStruct((n, d_out), jnp.float32),
    )(x, weight)

    bm = 200
    out, support_out = pl.pallas_call(
        _spmm_relu_kernel,
        grid=(n // bm,),
        in_specs=[
            pl.BlockSpec((bm, n), lambda i: (i, _I0)),
            pl.BlockSpec((n, d_out), lambda i: (_I0, _I0)),
        ],
        out_specs=[
            pl.BlockSpec((bm, d_out), lambda i: (i, _I0)),
            pl.BlockSpec((bm, n), lambda i: (i, _I0)),
        ],
        out_shape=[
            jax.ShapeDtypeStruct((n, d_out), jnp.float32),
            jax.ShapeDtypeStruct((n, n), jnp.float32),
        ],
        compiler_params=pltpu.CompilerParams(
            dimension_semantics=("arbitrary",),
        ),
    )(support, xw)

    return (out.astype(jnp.float64), support_out)


# speedup vs baseline: 1.0160x; 1.0160x over previous
import jax
import jax.numpy as jnp
import numpy as np
from jax.experimental import pallas as pl
from jax.experimental.pallas import tpu as pltpu

jax.config.update("jax_enable_x64", True)

_I0 = np.int32(0)


def _gcn_kernel(x_ref, w_ref, s_ref, o_ref, s_out_ref, xw_ref):
    @pl.when(pl.program_id(0) == 0)
    def _():
        xw_ref[...] = jnp.dot(x_ref[...], w_ref[...],
                              preferred_element_type=jnp.float32)

    s = s_ref[...]
    acc = jnp.dot(s, xw_ref[...], preferred_element_type=jnp.float32)
    o_ref[...] = jnp.maximum(acc, 0.0)
    s_out_ref[...] = s


def kernel(x, support, weight):
    n, d_in = x.shape
    d_out = weight.shape[1]

    bm = 200
    out, support_out = pl.pallas_call(
        _gcn_kernel,
        grid=(n // bm,),
        in_specs=[
            pl.BlockSpec((n, d_in), lambda i: (_I0, _I0)),
            pl.BlockSpec((d_in, d_out), lambda i: (_I0, _I0)),
            pl.BlockSpec((bm, n), lambda i: (i, _I0)),
        ],
        out_specs=[
            pl.BlockSpec((bm, d_out), lambda i: (i, _I0)),
            pl.BlockSpec((bm, n), lambda i: (i, _I0)),
        ],
        out_shape=[
            jax.ShapeDtypeStruct((n, d_out), jnp.float32),
            jax.ShapeDtypeStruct((n, n), jnp.float32),
        ],
        scratch_shapes=[pltpu.VMEM((n, d_out), jnp.float32)],
        compiler_params=pltpu.CompilerParams(
            dimension_semantics=("arbitrary",),
        ),
    )(x, weight, support)

    return (out.astype(jnp.float64), support_out)
